# baseline (speedup 1.0000x reference)
import functools

import jax
import jax.numpy as jnp
from jax.experimental import pallas as pl
from jax.experimental.pallas import tpu as pltpu

_TILE_B = 8
_K = 4


def _leaky(x):
    return jnp.where(x >= 0, x, 0.2 * x)


def _fused_kernel(n_a, nv, v_ref, ci_ref, msk_ref, f_ref, w1_ref, b1_ref,
                  w2_ref, b2_ref, w3_ref, b3_ref, cw_ref, cb_ref, g_ref,
                  bt_ref, o_ref, xs_ref, sum_ref, ssq_ref):
    i = pl.program_id(0)
    rows = v_ref.shape[0]
    nrows = n_a * rows
    f32 = jnp.float32
    bf = jnp.bfloat16

    @pl.when(i == 0)
    def _init():
        sum_ref[...] = jnp.zeros_like(sum_ref)
        ssq_ref[...] = jnp.zeros_like(ssq_ref)

    @pl.when(i < n_a)
    def _phase_a():
        Vb = v_ref[...]
        ci = ci_ref[...]
        msk = msk_ref[...]

        def dott_bf(a, b):
            return jax.lax.dot_general(a.astype(bf), b.astype(bf),
                                       (((1,), (1,)), ((), ())),
                                       preferred_element_type=f32)

        E = dott_bf(Vb, Vb)
        s2r = jnp.sum(Vb * Vb, axis=1, keepdims=True)
        s2c = jnp.transpose(s2r)
        D = ((-2.0 * E + s2c) + s2r) + msk

        inf = jnp.float32(jnp.inf)
        big = jnp.float32(1e9)
        ohs = []
        for _ in range(_K):
            mn = jnp.min(D, axis=1, keepdims=True)
            pos = jnp.where(D == mn, ci, big)
            first = jnp.min(pos, axis=1, keepdims=True)
            o = ci == first
            ohs.append(o.astype(f32))
            D = jnp.where(o, inf, D)

        ohsT = jnp.concatenate(ohs, axis=0)
        vks = jax.lax.dot_general(ohsT, Vb, (((1,), (0,)), ((), ())),
                                  precision=jax.lax.Precision.HIGHEST,
                                  preferred_element_type=f32)
        v0 = vks[0:rows]
        v0s = jnp.concatenate([v0, v0, v0, v0], axis=0)

        w1 = w1_ref[...]
        b1 = b1_ref[...]
        w2 = w2_ref[...]
        b2 = b2_ref[...]
        w3bf = w3_ref[...].astype(bf).astype(f32)
        b3 = b3_ref[0, 0]

        d = v0s - vks
        nr = jnp.sqrt(jnp.sum(d * d, axis=1, keepdims=True) + 1e-12)
        h = (dott_bf(v0s, w1[:, 0:3]) + dott_bf(vks, w1[:, 3:6])
             + dott_bf(d, w1[:, 6:9]) + dott_bf(nr, w1[:, 9:10]) + b1)
        h = _leaky(h)
        h = _leaky(dott_bf(h, w2) + b2)
        hbf = h.astype(bf).astype(f32)
        s = jnp.sum(hbf * w3bf, axis=1, keepdims=True) + b3

        A = (s[0:rows] * ohs[0] + s[rows:2 * rows] * ohs[1]
             + s[2 * rows:3 * rows] * ohs[2] + s[3 * rows:] * ohs[3])

        Fb = f_ref[...].astype(bf)
        X1 = jax.lax.dot_general(A.astype(bf), Fb, (((1,), (0,)), ((), ())),
                                 preferred_element_type=f32)
        X = jax.lax.dot_general(X1.astype(bf), cw_ref[...],
                                (((1,), (1,)), ((), ())),
                                preferred_element_type=f32) + cb_ref[...]
        xs_ref[pl.ds(i * rows, rows), :] = X.astype(bf)
        sum_ref[...] += jnp.sum(X, axis=0, keepdims=True)
        ssq_ref[...] += jnp.sum(X * X, axis=0, keepdims=True)

    @pl.when(i >= n_a)
    def _phase_b():
        j = i - n_a
        n = jnp.float32(nrows)
        mean = sum_ref[...] / n
        var = ssq_ref[...] / n - mean * mean
        rstd = jax.lax.rsqrt(var + 1e-5)
        x = xs_ref[pl.ds(j * rows, rows), :].astype(f32)
        y = (x - mean) * rstd * g_ref[...] + bt_ref[...]
        o_ref[...] = _leaky(y)


def kernel(F, V, R_w1, R_b1, R_w2, R_b2, R_w3, R_b3, conv_w, conv_b,
           bn_gamma, bn_beta):
    B, NV, D = F.shape
    rows = B * NV
    F2 = F.reshape(rows, D)
    V2 = V.reshape(rows, 3).astype(jnp.float32)
    tile = _TILE_B * NV
    n_a = rows // tile

    ci2 = jax.lax.broadcasted_iota(jnp.float32, (tile, tile), 1)
    rb = jax.lax.broadcasted_iota(jnp.int32, (tile, tile), 0) // NV
    cb = jax.lax.broadcasted_iota(jnp.int32, (tile, tile), 1) // NV
    msk2 = jnp.where(rb == cb, 0.0, jnp.inf).astype(jnp.float32)

    b1 = R_b1.reshape(1, 10)
    b2 = R_b2.reshape(1, 10)
    b3 = R_b3.reshape(1, 1)
    cwbf = conv_w.astype(jnp.bfloat16)
    cb_ = conv_b.reshape(1, D)
    g = bn_gamma.reshape(1, D)
    bt = bn_beta.reshape(1, D)

    def full(shp):
        return pl.BlockSpec(shp, lambda i: tuple(0 for _ in shp))

    last = n_a - 1
    out = pl.pallas_call(
        functools.partial(_fused_kernel, n_a, NV),
        grid=(2 * n_a,),
        in_specs=[
            pl.BlockSpec((tile, 3),
                         lambda i: (jnp.where(i < n_a, i, last), 0)),
            full((tile, tile)),
            full((tile, tile)),
            pl.BlockSpec((tile, D),
                         lambda i: (jnp.where(i < n_a, i, last), 0)),
            full((10, 10)), full((1, 10)), full((10, 10)), full((1, 10)),
            full((1, 10)),
            pl.BlockSpec((1, 1), lambda i: (0, 0), memory_space=pltpu.SMEM),
            full((D, D)), full((1, D)), full((1, D)), full((1, D)),
        ],
        out_specs=pl.BlockSpec((tile, D),
                               lambda i: (jnp.where(i < n_a, 0, i - n_a), 0)),
        out_shape=jax.ShapeDtypeStruct((rows, D), jnp.float32),
        scratch_shapes=[
            pltpu.VMEM((rows, D), jnp.bfloat16),
            pltpu.VMEM((1, D), jnp.float32),
            pltpu.VMEM((1, D), jnp.float32),
        ],
    )(V2, ci2, msk2, F2, R_w1, b1, R_w2, b2, R_w3, b3, cwbf, cb_, g, bt)
    return out.reshape(B, NV, D)


# speedup vs baseline: 1.2134x; 1.2134x over previous
import functools

import jax
import jax.numpy as jnp
from jax.experimental import pallas as pl
from jax.experimental.pallas import tpu as pltpu

_TILE_B = 16
_K = 4


def _leaky(x):
    return jnp.where(x >= 0, x, 0.2 * x)


def _graph_conv_kernel(nv, v_ref, ci_ref, msk_ref, f_ref, w1_ref,
                       b1_ref, w2_ref, b2_ref, w3_ref, b3_ref, cw_ref, cb_ref,
                       x_ref, sum_ref, ssq_ref):
    rows = v_ref.shape[0]
    f32 = jnp.float32
    bf = jnp.bfloat16
    Vb = v_ref[...]
    ci = ci_ref[...]
    msk = msk_ref[...]

    def dott_bf(a, b):
        return jax.lax.dot_general(a.astype(bf), b.astype(bf),
                                   (((1,), (1,)), ((), ())),
                                   preferred_element_type=f32)

    E = dott_bf(Vb, Vb)
    s2r = jnp.sum(Vb * Vb, axis=1, keepdims=True)
    s2c = jnp.transpose(s2r)
    D = ((-2.0 * E + s2c) + s2r) + msk

    inf = jnp.float32(jnp.inf)
    big = jnp.float32(1e9)
    ohs = []
    for _ in range(_K):
        mn = jnp.min(D, axis=1, keepdims=True)
        pos = jnp.where(D == mn, ci, big)
        first = jnp.min(pos, axis=1, keepdims=True)
        o = ci == first
        ohs.append(o.astype(f32))
        D = jnp.where(o, inf, D)

    ohsT = jnp.concatenate(ohs, axis=0)
    vks = jax.lax.dot_general(ohsT, Vb, (((1,), (0,)), ((), ())),
                              precision=jax.lax.Precision.HIGHEST,
                              preferred_element_type=f32)
    v0 = vks[0:rows]
    v0s = jnp.concatenate([v0, v0, v0, v0], axis=0)

    w1 = w1_ref[...]
    b1 = b1_ref[...]
    w2 = w2_ref[...]
    b2 = b2_ref[...]
    w3bf = w3_ref[...].astype(bf).astype(f32)
    b3 = b3_ref[0, 0]

    d = v0s - vks
    nr = jnp.sqrt(jnp.sum(d * d, axis=1, keepdims=True) + 1e-12)
    h = (dott_bf(v0s, w1[:, 0:3]) + dott_bf(vks, w1[:, 3:6])
         + dott_bf(d, w1[:, 6:9]) + dott_bf(nr, w1[:, 9:10]) + b1)
    h = _leaky(h)
    h = _leaky(dott_bf(h, w2) + b2)
    hbf = h.astype(bf).astype(f32)
    s = jnp.sum(hbf * w3bf, axis=1, keepdims=True) + b3

    A = (s[0:rows] * ohs[0] + s[rows:2 * rows] * ohs[1]
         + s[2 * rows:3 * rows] * ohs[2] + s[3 * rows:] * ohs[3])

    Fb = f_ref[...].astype(bf)
    X1 = jax.lax.dot_general(A.astype(bf), Fb, (((1,), (0,)), ((), ())),
                             preferred_element_type=f32)
    X = jax.lax.dot_general(X1.astype(bf), cw_ref[...],
                            (((1,), (1,)), ((), ())),
                            preferred_element_type=f32) + cb_ref[...]
    x_ref[...] = X.astype(bf)

    @pl.when(pl.program_id(0) == 0)
    def _init():
        sum_ref[...] = jnp.zeros_like(sum_ref)
        ssq_ref[...] = jnp.zeros_like(ssq_ref)

    sum_ref[...] += jnp.sum(X, axis=0, keepdims=True)
    ssq_ref[...] += jnp.sum(X * X, axis=0, keepdims=True)


def _bn_kernel(nrows, x_ref, sum_ref, ssq_ref, g_ref, bt_ref, o_ref):
    n = jnp.float32(nrows)
    mean = sum_ref[...] / n
    var = ssq_ref[...] / n - mean * mean
    rstd = jax.lax.rsqrt(var + 1e-5)
    y = (x_ref[...].astype(jnp.float32) - mean) * rstd * g_ref[...] + bt_ref[...]
    o_ref[...] = _leaky(y)


def kernel(F, V, R_w1, R_b1, R_w2, R_b2, R_w3, R_b3, conv_w, conv_b,
           bn_gamma, bn_beta):
    B, NV, D = F.shape
    rows = B * NV
    F2 = F.reshape(rows, D)
    V2 = V.reshape(rows, 3).astype(jnp.float32)
    tile = _TILE_B * NV
    grid = rows // tile

    ci2 = jax.lax.broadcasted_iota(jnp.float32, (tile, tile), 1)
    rb = jax.lax.broadcasted_iota(jnp.int32, (tile, tile), 0) // NV
    cb = jax.lax.broadcasted_iota(jnp.int32, (tile, tile), 1) // NV
    msk2 = jnp.where(rb == cb, 0.0, jnp.inf).astype(jnp.float32)

    b1 = R_b1.reshape(1, 10)
    b2 = R_b2.reshape(1, 10)
    b3 = R_b3.reshape(1, 1)
    cwbf = conv_w.astype(jnp.bfloat16)
    cb_ = conv_b.reshape(1, D)
    g = bn_gamma.reshape(1, D)
    bt = bn_beta.reshape(1, D)

    def full(shp):
        return pl.BlockSpec(shp, lambda i: tuple(0 for _ in shp))

    x, ssum, ssq = pl.pallas_call(
        functools.partial(_graph_conv_kernel, NV),
        grid=(grid,),
        in_specs=[
            pl.BlockSpec((tile, 3), lambda i: (i, 0)),
            full((tile, tile)),
            full((tile, tile)),
            pl.BlockSpec((tile, D), lambda i: (i, 0)),
            full((10, 10)), full((1, 10)), full((10, 10)), full((1, 10)),
            full((1, 10)),
            pl.BlockSpec((1, 1), lambda i: (0, 0), memory_space=pltpu.SMEM),
            full((D, D)), full((1, D)),
        ],
        out_specs=[
            pl.BlockSpec((tile, D), lambda i: (i, 0)),
            full((1, D)),
            full((1, D)),
        ],
        out_shape=[
            jax.ShapeDtypeStruct((rows, D), jnp.bfloat16),
            jax.ShapeDtypeStruct((1, D), jnp.float32),
            jax.ShapeDtypeStruct((1, D), jnp.float32),
        ],
    )(V2, ci2, msk2, F2, R_w1, b1, R_w2, b2, R_w3, b3, cwbf, cb_)

    tile2 = 2048
    grid2 = rows // tile2
    out = pl.pallas_call(
        functools.partial(_bn_kernel, rows),
        grid=(grid2,),
        in_specs=[
            pl.BlockSpec((tile2, D), lambda i: (i, 0)),
            full((1, D)), full((1, D)), full((1, D)), full((1, D)),
        ],
        out_specs=pl.BlockSpec((tile2, D), lambda i: (i, 0)),
        out_shape=jax.ShapeDtypeStruct((rows, D), jnp.float32),
    )(x, ssum, ssq, g, bt)
    return out.reshape(B, NV, D)
